# baseline (speedup 1.0000x reference)
import functools

import jax
import jax.numpy as jnp
from jax import lax
from jax.experimental import pallas as pl
from jax.experimental.pallas import tpu as pltpu
from jax.experimental.pallas import tpu_sc as plsc

H = 256
W = 256
NEAR = 0.1
FAR = 100.0
REC = 32
NSUB = 16
NCORE = 2
NPIX = 16 * W


def _raster_body(rec_hbm, feat_out, fidx_out, depth_out,
                 fv, xt, depth_v, fidx_v, f0_v, f1_v, f2_v, f3_v):
    b = lax.axis_index("c")
    band = lax.axis_index("s")
    F = fv.shape[0]

    pltpu.sync_copy(rec_hbm.at[b], fv)

    far16 = jnp.full((16,), FAR, jnp.float32)
    neg16 = jnp.full((16,), -1, jnp.int32)
    zero16 = jnp.zeros((16,), jnp.float32)
    lane = lax.iota(jnp.int32, 16)

    def init_body(j, carry):
        c = pl.multiple_of(j * 16, 16)
        depth_v[pl.ds(c, 16)] = far16
        fidx_v[pl.ds(c, 16)] = neg16
        f0_v[pl.ds(c, 16)] = zero16
        f1_v[pl.ds(c, 16)] = zero16
        f2_v[pl.ds(c, 16)] = zero16
        return carry

    lax.fori_loop(0, NPIX // 16, init_body, 0)

    def xt_body(j, carry):
        c = pl.multiple_of(j * 16, 16)
        xg = ((c + lane).astype(jnp.float32) * 2.0 + (1.0 - W)) * (1.0 / W)
        xt[pl.ds(c, 16)] = xg
        return carry

    lax.fori_loop(0, W // 16, xt_body, 0)

    def face_body(f, carry):
        row0 = fv[f, pl.ds(0, 16)]
        row1 = fv[f, pl.ds(16, 16)]
        r_lo = row0[11].astype(jnp.int32)
        r_hi = row0[12].astype(jnp.int32)
        c_lo = row0[13].astype(jnp.int32)
        ng = row0[14].astype(jnp.int32)
        m = lax.rem(band - r_lo, 16)
        m = m + jnp.where(m < 0, 16, 0)
        r0 = r_lo + m
        nk = (r_hi - r0 + 15) >> 4
        rl0 = (r0 - band) >> 4

        @pl.when(nk > 0)
        def _process():
            a0 = row0[0]
            b0 = row0[1]
            a1 = row0[2]
            b1 = row0[3]
            x2 = row0[4]
            y2 = row0[5]
            sgn = row0[6]
            rsafe = row0[7]
            rq0 = row0[8]
            rq1 = row0[9]
            rz2 = row0[10]
            t00 = row1[0]
            t01 = row1[1]
            t02 = row1[2]
            t10 = row1[3]
            t11 = row1[4]
            t12 = row1[5]
            t20 = row1[6]
            t21 = row1[7]
            t22 = row1[8]
            fvec = jnp.broadcast_to(f.astype(jnp.int32), (16,))

            @plsc.parallel_loop(0, ng)
            def g_body(gi):
                g = pl.multiple_of(c_lo + gi * 16, 16)
                X = xt[pl.ds(g, 16)]
                xm = X - x2
                pa = a0 * xm
                pb = a1 * xm

                @plsc.parallel_loop(0, nk, unroll=2)
                def r_body(k):
                    r = r0 + k * 16
                    rl = rl0 + k
                    base = pl.multiple_of(rl * W + g, 16)
                    yi = (r.astype(jnp.float32) * 2.0 + (1.0 - H)) * (1.0 / H)
                    yd = yi - y2
                    num0 = pa + b0 * yd
                    num1 = pb + b1 * yd
                    w0 = num0 * rsafe
                    w1 = num1 * rsafe
                    w2 = 1.0 - w0 - w1
                    inside = ((num0 * sgn >= 0.0) & (num1 * sgn >= 0.0)
                              & (w2 >= 0.0))
                    u0 = num0 * rq0
                    u1 = num1 * rq1
                    u2 = w2 * rz2
                    inv_z = u0 + u1 + u2
                    zp = 1.0 / inv_z
                    dcur = depth_v[pl.ds(base, 16)]
                    valid = inside & (zp > NEAR) & (zp < dcur)
                    wc0 = u0 * zp
                    wc1 = u1 * zp
                    wc2 = u2 * zp
                    depth_v[pl.ds(base, 16)] = jnp.where(valid, zp, dcur)
                    fidx_v[pl.ds(base, 16)] = jnp.where(
                        valid, fvec, fidx_v[pl.ds(base, 16)])
                    f0_v[pl.ds(base, 16)] = jnp.where(
                        valid, wc0 * t00 + wc1 * t10 + wc2 * t20,
                        f0_v[pl.ds(base, 16)])
                    f1_v[pl.ds(base, 16)] = jnp.where(
                        valid, wc0 * t01 + wc1 * t11 + wc2 * t21,
                        f1_v[pl.ds(base, 16)])
                    f2_v[pl.ds(base, 16)] = jnp.where(
                        valid, wc0 * t02 + wc1 * t12 + wc2 * t22,
                        f2_v[pl.ds(base, 16)])

        return carry

    lax.fori_loop(0, F, face_body, 0)

    def mask_body(j, carry):
        c = pl.multiple_of(j * 16, 16)
        fx = fidx_v[pl.ds(c, 16)]
        f3_v[pl.ds(c, 16)] = jnp.where(fx >= 0, 1.0, 0.0).astype(jnp.float32)
        return carry

    lax.fori_loop(0, NPIX // 16, mask_body, 0)

    for rl in range(16):
        r = band + rl * 16
        s = rl * W
        pltpu.sync_copy(f0_v.at[pl.ds(s, W)], feat_out.at[b, 0, r])
        pltpu.sync_copy(f1_v.at[pl.ds(s, W)], feat_out.at[b, 1, r])
        pltpu.sync_copy(f2_v.at[pl.ds(s, W)], feat_out.at[b, 2, r])
        pltpu.sync_copy(f3_v.at[pl.ds(s, W)], feat_out.at[b, 3, r])
        pltpu.sync_copy(fidx_v.at[pl.ds(s, W)], fidx_out.at[b, r])
        pltpu.sync_copy(depth_v.at[pl.ds(s, W)], depth_out.at[b, r])


@functools.lru_cache(maxsize=None)
def _make_raster(B, F, T):
    mesh = plsc.VectorSubcoreMesh(
        core_axis_name="c", subcore_axis_name="s",
        num_cores=NCORE, num_subcores=NSUB)
    return pl.kernel(
        _raster_body,
        out_type=[
            jax.ShapeDtypeStruct((B, T + 1, H, W), jnp.float32),
            jax.ShapeDtypeStruct((B, H, W), jnp.int32),
            jax.ShapeDtypeStruct((B, H, W), jnp.float32),
        ],
        mesh=mesh,
        scratch_types=[
            pltpu.VMEM((F, REC), jnp.float32),
            pltpu.VMEM((W,), jnp.float32),
            pltpu.VMEM((NPIX,), jnp.float32),
            pltpu.VMEM((NPIX,), jnp.int32),
            pltpu.VMEM((NPIX,), jnp.float32),
            pltpu.VMEM((NPIX,), jnp.float32),
            pltpu.VMEM((NPIX,), jnp.float32),
            pltpu.VMEM((NPIX,), jnp.float32),
        ],
    )


def kernel(faces, textures):
    B, F = faces.shape[0], faces.shape[1]
    T = textures.shape[-1]
    faces = faces.astype(jnp.float32)
    x0 = faces[:, :, 0, 0]
    y0 = faces[:, :, 0, 1]
    z0 = faces[:, :, 0, 2]
    x1 = faces[:, :, 1, 0]
    y1 = faces[:, :, 1, 1]
    z1 = faces[:, :, 1, 2]
    x2 = faces[:, :, 2, 0]
    y2 = faces[:, :, 2, 1]
    z2 = faces[:, :, 2, 2]
    a0 = y1 - y2
    b0 = x2 - x1
    a1 = y2 - y0
    b1 = x0 - x2
    denom = a0 * (x0 - x2) + b0 * (y0 - y2)
    ok = jnp.abs(denom) > 1e-10
    safe = jnp.where(ok, denom, 1.0)
    sgn = jnp.where(safe >= 0, 1.0, -1.0).astype(jnp.float32)
    rsafe = 1.0 / safe
    rq0 = 1.0 / (safe * z0)
    rq1 = 1.0 / (safe * z1)
    rz2 = 1.0 / z2
    pad = 2.0 / W
    xmin = jnp.minimum(jnp.minimum(x0, x1), x2) - pad
    xmax = jnp.maximum(jnp.maximum(x0, x1), x2) + pad
    ymin = jnp.minimum(jnp.minimum(y0, y1), y2) - pad
    ymax = jnp.maximum(jnp.maximum(y0, y1), y2) + pad
    umin = (xmin * W + (W - 1.0)) * 0.5
    umax = (xmax * W + (W - 1.0)) * 0.5
    vmin = (ymin * H + (H - 1.0)) * 0.5
    vmax = (ymax * H + (H - 1.0)) * 0.5
    empty = ((vmax < 0.0) | (vmin > H - 1.0) | (umax < 0.0)
             | (umin > W - 1.0) | ~ok)
    r_lo = jnp.clip(vmin, 0.0, H - 1.0).astype(jnp.int32)
    r_hi = jnp.clip(vmax, 0.0, H - 1.0).astype(jnp.int32) + 1
    c_lo = jnp.clip(umin, 0.0, W - 1.0).astype(jnp.int32)
    c_lo = (c_lo >> 4) << 4
    c_hi = jnp.clip(umax, 0.0, W - 1.0).astype(jnp.int32) + 1
    ng = (c_hi - c_lo + 15) >> 4
    r_lo = jnp.where(empty, 0, r_lo)
    r_hi = jnp.where(empty, 0, r_hi)
    f32 = lambda v: v.astype(jnp.float32)
    tex = textures.astype(jnp.float32).reshape(B, F, 3 * T)
    rec = jnp.stack([a0, b0, a1, b1, x2, y2, sgn, rsafe, rq0, rq1, rz2,
                     f32(r_lo), f32(r_hi), f32(c_lo), f32(ng)], axis=-1)
    rec = jnp.concatenate(
        [rec, jnp.zeros((B, F, 1), jnp.float32), tex,
         jnp.zeros((B, F, REC - 25), jnp.float32)],
        axis=-1)
    feature, fidx, depth = _make_raster(B, F, T)(rec)
    return feature, fidx, depth


# speedup vs baseline: 1.0292x; 1.0292x over previous
import functools

import jax
import jax.numpy as jnp
from jax import lax
from jax.experimental import pallas as pl
from jax.experimental.pallas import tpu as pltpu
from jax.experimental.pallas import tpu_sc as plsc

H = 256
W = 256
NEAR = 0.1
FAR = 100.0
REC = 32
NSUB = 16
NCORE = 2
NPIX = 16 * W


def _raster_body(rec_hbm, feat_out, fidx_out, depth_out,
                 fv, xt, depth_v, fidx_v, f0_v, f1_v, f2_v, f3_v):
    b = lax.axis_index("c")
    band = lax.axis_index("s")
    F = fv.shape[0]

    pltpu.sync_copy(rec_hbm.at[b], fv)

    far16 = jnp.full((16,), FAR, jnp.float32)
    neg16 = jnp.full((16,), -1, jnp.int32)
    zero16 = jnp.zeros((16,), jnp.float32)
    lane = lax.iota(jnp.int32, 16)

    def init_body(j, carry):
        c = pl.multiple_of(j * 16, 16)
        depth_v[pl.ds(c, 16)] = far16
        fidx_v[pl.ds(c, 16)] = neg16
        f0_v[pl.ds(c, 16)] = zero16
        f1_v[pl.ds(c, 16)] = zero16
        f2_v[pl.ds(c, 16)] = zero16
        return carry

    lax.fori_loop(0, NPIX // 16, init_body, 0)

    def xt_body(j, carry):
        c = pl.multiple_of(j * 16, 16)
        xg = ((c + lane).astype(jnp.float32) * 2.0 + (1.0 - W)) * (1.0 / W)
        xt[pl.ds(c, 16)] = xg
        return carry

    lax.fori_loop(0, W // 16, xt_body, 0)

    def face_body(f, carry):
        row0 = fv[f, pl.ds(0, 16)]
        row1 = fv[f, pl.ds(16, 16)]
        r_lo = row0[11].astype(jnp.int32)
        r_hi = row0[12].astype(jnp.int32)
        c_lo = row0[13].astype(jnp.int32)
        ng = row0[14].astype(jnp.int32)
        m = lax.rem(band - r_lo, 16)
        m = m + jnp.where(m < 0, 16, 0)
        r0 = r_lo + m
        nk = (r_hi - r0 + 15) >> 4
        rl0 = (r0 - band) >> 4

        @pl.when(nk > 0)
        def _process():
            a0 = row0[0]
            b0 = row0[1]
            a1 = row0[2]
            b1 = row0[3]
            x2 = row0[4]
            y2 = row0[5]
            sgn = row0[6]
            rsafe = row0[7]
            rq0 = row0[8]
            rq1 = row0[9]
            rz2 = row0[10]
            y2v = jnp.broadcast_to(y2, (16,))
            t00 = row1[0]
            t01 = row1[1]
            t02 = row1[2]
            t10 = row1[3]
            t11 = row1[4]
            t12 = row1[5]
            t20 = row1[6]
            t21 = row1[7]
            t22 = row1[8]
            fvec = jnp.broadcast_to(f.astype(jnp.int32), (16,))

            @plsc.parallel_loop(0, ng)
            def g_body(gi):
                g = pl.multiple_of(c_lo + gi * 16, 16)
                X = xt[pl.ds(g, 16)]
                xm = X - x2
                pa = a0 * xm
                pb = a1 * xm

                @plsc.parallel_loop(0, nk, unroll=2)
                def r_body(k):
                    r = r0 + k * 16
                    rl = rl0 + k
                    base = pl.multiple_of(rl * W + g, 16)
                    yi = (r.astype(jnp.float32) * 2.0 + (1.0 - H)) * (1.0 / H)
                    yd = yi - y2v
                    num0 = pa + b0 * yd
                    num1 = pb + b1 * yd
                    w0 = num0 * rsafe
                    w1 = num1 * rsafe
                    w2 = 1.0 - w0 - w1
                    inside = ((num0 * sgn >= 0.0) & (num1 * sgn >= 0.0)
                              & (w2 >= 0.0))
                    u0 = num0 * rq0
                    u1 = num1 * rq1
                    u2 = w2 * rz2
                    inv_z = u0 + u1 + u2
                    zp = 1.0 / inv_z
                    dcur = depth_v[pl.ds(base, 16)]
                    valid = inside & (zp > NEAR) & (zp < dcur)
                    wc0 = u0 * zp
                    wc1 = u1 * zp
                    wc2 = u2 * zp
                    depth_v[pl.ds(base, 16)] = jnp.where(valid, zp, dcur)
                    fidx_v[pl.ds(base, 16)] = jnp.where(
                        valid, fvec, fidx_v[pl.ds(base, 16)])
                    f0_v[pl.ds(base, 16)] = jnp.where(
                        valid, wc0 * t00 + wc1 * t10 + wc2 * t20,
                        f0_v[pl.ds(base, 16)])
                    f1_v[pl.ds(base, 16)] = jnp.where(
                        valid, wc0 * t01 + wc1 * t11 + wc2 * t21,
                        f1_v[pl.ds(base, 16)])
                    f2_v[pl.ds(base, 16)] = jnp.where(
                        valid, wc0 * t02 + wc1 * t12 + wc2 * t22,
                        f2_v[pl.ds(base, 16)])

        return carry

    lax.fori_loop(0, F, face_body, 0)

    def mask_body(j, carry):
        c = pl.multiple_of(j * 16, 16)
        fx = fidx_v[pl.ds(c, 16)]
        f3_v[pl.ds(c, 16)] = jnp.where(fx >= 0, 1.0, 0.0).astype(jnp.float32)
        return carry

    lax.fori_loop(0, NPIX // 16, mask_body, 0)

    for rl in range(16):
        r = band + rl * 16
        s = rl * W
        pltpu.sync_copy(f0_v.at[pl.ds(s, W)], feat_out.at[b, 0, r])
        pltpu.sync_copy(f1_v.at[pl.ds(s, W)], feat_out.at[b, 1, r])
        pltpu.sync_copy(f2_v.at[pl.ds(s, W)], feat_out.at[b, 2, r])
        pltpu.sync_copy(f3_v.at[pl.ds(s, W)], feat_out.at[b, 3, r])
        pltpu.sync_copy(fidx_v.at[pl.ds(s, W)], fidx_out.at[b, r])
        pltpu.sync_copy(depth_v.at[pl.ds(s, W)], depth_out.at[b, r])


@functools.lru_cache(maxsize=None)
def _make_raster(B, F, T):
    mesh = plsc.VectorSubcoreMesh(
        core_axis_name="c", subcore_axis_name="s",
        num_cores=NCORE, num_subcores=NSUB)
    return pl.kernel(
        _raster_body,
        out_type=[
            jax.ShapeDtypeStruct((B, T + 1, H, W), jnp.float32),
            jax.ShapeDtypeStruct((B, H, W), jnp.int32),
            jax.ShapeDtypeStruct((B, H, W), jnp.float32),
        ],
        mesh=mesh,
        scratch_types=[
            pltpu.VMEM((F, REC), jnp.float32),
            pltpu.VMEM((W,), jnp.float32),
            pltpu.VMEM((NPIX,), jnp.float32),
            pltpu.VMEM((NPIX,), jnp.int32),
            pltpu.VMEM((NPIX,), jnp.float32),
            pltpu.VMEM((NPIX,), jnp.float32),
            pltpu.VMEM((NPIX,), jnp.float32),
            pltpu.VMEM((NPIX,), jnp.float32),
        ],
    )


def kernel(faces, textures):
    B, F = faces.shape[0], faces.shape[1]
    T = textures.shape[-1]
    faces = faces.astype(jnp.float32)
    x0 = faces[:, :, 0, 0]
    y0 = faces[:, :, 0, 1]
    z0 = faces[:, :, 0, 2]
    x1 = faces[:, :, 1, 0]
    y1 = faces[:, :, 1, 1]
    z1 = faces[:, :, 1, 2]
    x2 = faces[:, :, 2, 0]
    y2 = faces[:, :, 2, 1]
    z2 = faces[:, :, 2, 2]
    a0 = y1 - y2
    b0 = x2 - x1
    a1 = y2 - y0
    b1 = x0 - x2
    denom = a0 * (x0 - x2) + b0 * (y0 - y2)
    ok = jnp.abs(denom) > 1e-10
    safe = jnp.where(ok, denom, 1.0)
    sgn = jnp.where(safe >= 0, 1.0, -1.0).astype(jnp.float32)
    rsafe = 1.0 / safe
    rq0 = 1.0 / (safe * z0)
    rq1 = 1.0 / (safe * z1)
    rz2 = 1.0 / z2
    pad = 2.0 / W
    xmin = jnp.minimum(jnp.minimum(x0, x1), x2) - pad
    xmax = jnp.maximum(jnp.maximum(x0, x1), x2) + pad
    ymin = jnp.minimum(jnp.minimum(y0, y1), y2) - pad
    ymax = jnp.maximum(jnp.maximum(y0, y1), y2) + pad
    umin = (xmin * W + (W - 1.0)) * 0.5
    umax = (xmax * W + (W - 1.0)) * 0.5
    vmin = (ymin * H + (H - 1.0)) * 0.5
    vmax = (ymax * H + (H - 1.0)) * 0.5
    empty = ((vmax < 0.0) | (vmin > H - 1.0) | (umax < 0.0)
             | (umin > W - 1.0) | ~ok)
    r_lo = jnp.clip(vmin, 0.0, H - 1.0).astype(jnp.int32)
    r_hi = jnp.clip(vmax, 0.0, H - 1.0).astype(jnp.int32) + 1
    c_lo = jnp.clip(umin, 0.0, W - 1.0).astype(jnp.int32)
    c_lo = (c_lo >> 4) << 4
    c_hi = jnp.clip(umax, 0.0, W - 1.0).astype(jnp.int32) + 1
    ng = (c_hi - c_lo + 15) >> 4
    r_lo = jnp.where(empty, 0, r_lo)
    r_hi = jnp.where(empty, 0, r_hi)
    f32 = lambda v: v.astype(jnp.float32)
    tex = textures.astype(jnp.float32).reshape(B, F, 3 * T)
    rec = jnp.stack([a0, b0, a1, b1, x2, y2, sgn, rsafe, rq0, rq1, rz2,
                     f32(r_lo), f32(r_hi), f32(c_lo), f32(ng)], axis=-1)
    rec = jnp.concatenate(
        [rec, jnp.zeros((B, F, 1), jnp.float32), tex,
         jnp.zeros((B, F, REC - 25), jnp.float32)],
        axis=-1)
    feature, fidx, depth = _make_raster(B, F, T)(rec)
    return feature, fidx, depth


# speedup vs baseline: 1.3874x; 1.3480x over previous
import functools

import jax
import jax.numpy as jnp
from jax import lax
from jax.experimental import pallas as pl
from jax.experimental.pallas import tpu as pltpu
from jax.experimental.pallas import tpu_sc as plsc

H = 256
W = 256
NEAR = 0.1
FAR = 100.0
REC = 32
NSUB = 16
NCORE = 2
NPIX = 16 * W


def _raster_body(rec_hbm, feat_out, fidx_out, depth_out,
                 fv, xt, depth_v, fidx_v, f0_v, f1_v, f2_v, f3_v):
    b = lax.axis_index("c")
    band = lax.axis_index("s")
    F = fv.shape[0]

    pltpu.sync_copy(rec_hbm.at[b], fv)

    far16 = jnp.full((16,), FAR, jnp.float32)
    neg16 = jnp.full((16,), -1, jnp.int32)
    zero16 = jnp.zeros((16,), jnp.float32)
    lane = lax.iota(jnp.int32, 16)

    def init_body(j, carry):
        rl = j >> 4
        c = pl.multiple_of((j & 15) * 16, 16)
        depth_v[rl, pl.ds(c, 16)] = far16
        fidx_v[rl, pl.ds(c, 16)] = neg16
        f0_v[rl, pl.ds(c, 16)] = zero16
        f1_v[rl, pl.ds(c, 16)] = zero16
        f2_v[rl, pl.ds(c, 16)] = zero16
        return carry

    lax.fori_loop(0, NPIX // 16, init_body, 0)

    def xt_body(j, carry):
        c = pl.multiple_of(j * 16, 16)
        xg = ((c + lane).astype(jnp.float32) * 2.0 + (1.0 - W)) * (1.0 / W)
        xt[pl.ds(c, 16)] = xg
        return carry

    lax.fori_loop(0, W // 16, xt_body, 0)

    band0 = band * 16

    def face_body(f, carry):
        row0 = fv[f, pl.ds(0, 16)]
        row1 = fv[f, pl.ds(16, 16)]
        r_lo = row0[11].astype(jnp.int32)
        r_hi = row0[12].astype(jnp.int32)
        c_lo = row0[13].astype(jnp.int32)
        ng = row0[14].astype(jnp.int32)
        r0 = jnp.maximum(r_lo, band0)
        nk = jnp.minimum(r_hi, band0 + 16) - r0
        rl0 = r0 - band0

        @pl.when(nk > 0)
        def _process():
            a0 = row0[0]
            b0 = row0[1]
            a1 = row0[2]
            b1 = row0[3]
            x2 = row0[4]
            y2 = row0[5]
            sgn = row0[6]
            rsafe = row0[7]
            rq0 = row0[8]
            rq1 = row0[9]
            rz2 = row0[10]
            y2v = jnp.broadcast_to(y2, (16,))
            t00 = row1[0]
            t01 = row1[1]
            t02 = row1[2]
            t10 = row1[3]
            t11 = row1[4]
            t12 = row1[5]
            t20 = row1[6]
            t21 = row1[7]
            t22 = row1[8]
            fvec = jnp.broadcast_to(f.astype(jnp.int32), (16,))

            @plsc.parallel_loop(0, ng)
            def g_body(gi):
                g = pl.multiple_of(c_lo + gi * 16, 16)
                X = xt[pl.ds(g, 16)]
                xm = X - x2
                pa = a0 * xm
                pb = a1 * xm

                @plsc.parallel_loop(0, nk, unroll=2)
                def r_body(k):
                    r = r0 + k
                    rl = rl0 + k
                    yi = (r.astype(jnp.float32) * 2.0 + (1.0 - H)) * (1.0 / H)
                    yd = yi - y2v
                    num0 = pa + b0 * yd
                    num1 = pb + b1 * yd
                    w0 = num0 * rsafe
                    w1 = num1 * rsafe
                    w2 = 1.0 - w0 - w1
                    inside = ((num0 * sgn >= 0.0) & (num1 * sgn >= 0.0)
                              & (w2 >= 0.0))
                    u0 = num0 * rq0
                    u1 = num1 * rq1
                    u2 = w2 * rz2
                    inv_z = u0 + u1 + u2
                    zp = 1.0 / inv_z
                    dcur = depth_v[rl, pl.ds(g, 16)]
                    valid = inside & (zp > NEAR) & (zp < dcur)
                    wc0 = u0 * zp
                    wc1 = u1 * zp
                    wc2 = u2 * zp
                    depth_v[rl, pl.ds(g, 16)] = jnp.where(valid, zp, dcur)
                    fidx_v[rl, pl.ds(g, 16)] = jnp.where(
                        valid, fvec, fidx_v[rl, pl.ds(g, 16)])
                    f0_v[rl, pl.ds(g, 16)] = jnp.where(
                        valid, wc0 * t00 + wc1 * t10 + wc2 * t20,
                        f0_v[rl, pl.ds(g, 16)])
                    f1_v[rl, pl.ds(g, 16)] = jnp.where(
                        valid, wc0 * t01 + wc1 * t11 + wc2 * t21,
                        f1_v[rl, pl.ds(g, 16)])
                    f2_v[rl, pl.ds(g, 16)] = jnp.where(
                        valid, wc0 * t02 + wc1 * t12 + wc2 * t22,
                        f2_v[rl, pl.ds(g, 16)])

        return carry

    lax.fori_loop(0, F, face_body, 0)

    def mask_body(j, carry):
        rl = j >> 4
        c = pl.multiple_of((j & 15) * 16, 16)
        fx = fidx_v[rl, pl.ds(c, 16)]
        f3_v[rl, pl.ds(c, 16)] = jnp.where(fx >= 0, 1.0, 0.0).astype(
            jnp.float32)
        return carry

    lax.fori_loop(0, NPIX // 16, mask_body, 0)

    pltpu.sync_copy(f0_v, feat_out.at[b, 0, pl.ds(band0, 16)])
    pltpu.sync_copy(f1_v, feat_out.at[b, 1, pl.ds(band0, 16)])
    pltpu.sync_copy(f2_v, feat_out.at[b, 2, pl.ds(band0, 16)])
    pltpu.sync_copy(f3_v, feat_out.at[b, 3, pl.ds(band0, 16)])
    pltpu.sync_copy(fidx_v, fidx_out.at[b, pl.ds(band0, 16)])
    pltpu.sync_copy(depth_v, depth_out.at[b, pl.ds(band0, 16)])


@functools.lru_cache(maxsize=None)
def _make_raster(B, F, T):
    mesh = plsc.VectorSubcoreMesh(
        core_axis_name="c", subcore_axis_name="s",
        num_cores=NCORE, num_subcores=NSUB)
    return pl.kernel(
        _raster_body,
        out_type=[
            jax.ShapeDtypeStruct((B, T + 1, H, W), jnp.float32),
            jax.ShapeDtypeStruct((B, H, W), jnp.int32),
            jax.ShapeDtypeStruct((B, H, W), jnp.float32),
        ],
        mesh=mesh,
        scratch_types=[
            pltpu.VMEM((F, REC), jnp.float32),
            pltpu.VMEM((W,), jnp.float32),
            pltpu.VMEM((16, W), jnp.float32),
            pltpu.VMEM((16, W), jnp.int32),
            pltpu.VMEM((16, W), jnp.float32),
            pltpu.VMEM((16, W), jnp.float32),
            pltpu.VMEM((16, W), jnp.float32),
            pltpu.VMEM((16, W), jnp.float32),
        ],
    )


def kernel(faces, textures):
    B, F = faces.shape[0], faces.shape[1]
    T = textures.shape[-1]
    faces = faces.astype(jnp.float32)
    x0 = faces[:, :, 0, 0]
    y0 = faces[:, :, 0, 1]
    z0 = faces[:, :, 0, 2]
    x1 = faces[:, :, 1, 0]
    y1 = faces[:, :, 1, 1]
    z1 = faces[:, :, 1, 2]
    x2 = faces[:, :, 2, 0]
    y2 = faces[:, :, 2, 1]
    z2 = faces[:, :, 2, 2]
    a0 = y1 - y2
    b0 = x2 - x1
    a1 = y2 - y0
    b1 = x0 - x2
    denom = a0 * (x0 - x2) + b0 * (y0 - y2)
    ok = jnp.abs(denom) > 1e-10
    safe = jnp.where(ok, denom, 1.0)
    sgn = jnp.where(safe >= 0, 1.0, -1.0).astype(jnp.float32)
    rsafe = 1.0 / safe
    rq0 = 1.0 / (safe * z0)
    rq1 = 1.0 / (safe * z1)
    rz2 = 1.0 / z2
    pad = 2.0 / W
    xmin = jnp.minimum(jnp.minimum(x0, x1), x2) - pad
    xmax = jnp.maximum(jnp.maximum(x0, x1), x2) + pad
    ymin = jnp.minimum(jnp.minimum(y0, y1), y2) - pad
    ymax = jnp.maximum(jnp.maximum(y0, y1), y2) + pad
    umin = (xmin * W + (W - 1.0)) * 0.5
    umax = (xmax * W + (W - 1.0)) * 0.5
    vmin = (ymin * H + (H - 1.0)) * 0.5
    vmax = (ymax * H + (H - 1.0)) * 0.5
    empty = ((vmax < 0.0) | (vmin > H - 1.0) | (umax < 0.0)
             | (umin > W - 1.0) | ~ok)
    r_lo = jnp.clip(vmin, 0.0, H - 1.0).astype(jnp.int32)
    r_hi = jnp.clip(vmax, 0.0, H - 1.0).astype(jnp.int32) + 1
    c_lo = jnp.clip(umin, 0.0, W - 1.0).astype(jnp.int32)
    c_lo = (c_lo >> 4) << 4
    c_hi = jnp.clip(umax, 0.0, W - 1.0).astype(jnp.int32) + 1
    ng = (c_hi - c_lo + 15) >> 4
    r_lo = jnp.where(empty, 0, r_lo)
    r_hi = jnp.where(empty, 0, r_hi)
    f32 = lambda v: v.astype(jnp.float32)
    tex = textures.astype(jnp.float32).reshape(B, F, 3 * T)
    rec = jnp.stack([a0, b0, a1, b1, x2, y2, sgn, rsafe, rq0, rq1, rz2,
                     f32(r_lo), f32(r_hi), f32(c_lo), f32(ng)], axis=-1)
    rec = jnp.concatenate(
        [rec, jnp.zeros((B, F, 1), jnp.float32), tex,
         jnp.zeros((B, F, REC - 25), jnp.float32)],
        axis=-1)
    feature, fidx, depth = _make_raster(B, F, T)(rec)
    return feature, fidx, depth
